# baseline (speedup 1.0000x reference)
import functools

import jax
import jax.numpy as jnp
from jax import lax
from jax.experimental import pallas as pl
from jax.experimental.pallas import tpu as pltpu
from jax.experimental.pallas import tpu_sc as plsc

B, S, D = 1024, 26, 128
P = S * (S - 1) // 2
NC, NS = 2, 16
NW = NC * NS
B_PER_W = B // NW
NVR = D // 16

_mesh = plsc.VectorSubcoreMesh(core_axis_name="c", subcore_axis_name="s")


@functools.partial(
    pl.kernel,
    mesh=_mesh,
    out_type=jax.ShapeDtypeStruct((B, P, D), jnp.float32),
    scratch_types=[
        pltpu.VMEM((S, D), jnp.float32),
        pltpu.VMEM((P, D), jnp.float32),
    ],
)
def _pairwise(x_hbm, out_hbm, x_v, out_v):
    wid = lax.axis_index("s") * NC + lax.axis_index("c")
    base = wid * B_PER_W

    def batch_body(g, carry):
        b = base + g
        pltpu.sync_copy(x_hbm.at[b], x_v)
        for i in range(S - 1):
            row_i = [x_v[i, pl.ds(k * 16, 16)] for k in range(NVR)]
            off = 25 * i - (i * (i - 1)) // 2 - i - 1

            def j_body(j, c, row_i=row_i, off=off):
                for k in range(NVR):
                    out_v[off + j, pl.ds(k * 16, 16)] = (
                        row_i[k] * x_v[j, pl.ds(k * 16, 16)]
                    )
                return c

            lax.fori_loop(i + 1, S, j_body, 0)
        pltpu.sync_copy(out_v, out_hbm.at[b])
        return carry

    lax.fori_loop(0, B_PER_W, batch_body, 0)


def kernel(x):
    return _pairwise(x)


# speedup vs baseline: 1.1208x; 1.1208x over previous
import functools

import jax
import jax.numpy as jnp
from jax import lax
from jax.experimental import pallas as pl
from jax.experimental.pallas import tpu as pltpu
from jax.experimental.pallas import tpu_sc as plsc

B, S, D = 1024, 26, 128
P = S * (S - 1) // 2
NC, NS = 2, 16
NW = NC * NS
B_PER_W = B // NW
NVR = D // 16

_mesh = plsc.VectorSubcoreMesh(core_axis_name="c", subcore_axis_name="s")


@functools.partial(
    pl.kernel,
    mesh=_mesh,
    out_type=jax.ShapeDtypeStruct((B, P, D), jnp.float32),
    scratch_types=[
        pltpu.VMEM((S, D), jnp.float32),
        pltpu.VMEM((P, D), jnp.float32),
        pltpu.VMEM((P, D), jnp.float32),
        pltpu.SemaphoreType.DMA,
        pltpu.SemaphoreType.DMA,
    ],
)
def _pairwise(x_hbm, out_hbm, x_v, out0, out1, sem0, sem1):
    wid = lax.axis_index("s") * NC + lax.axis_index("c")
    base = wid * B_PER_W

    def compute(buf):
        for i in range(S - 1):
            row_i = [x_v[i, pl.ds(k * 16, 16)] for k in range(NVR)]
            off = 25 * i - (i * (i - 1)) // 2 - i - 1

            def j_body(j, c, row_i=row_i, off=off):
                for k in range(NVR):
                    buf[off + j, pl.ds(k * 16, 16)] = (
                        row_i[k] * x_v[j, pl.ds(k * 16, 16)]
                    )
                return c

            lax.fori_loop(i + 1, S, j_body, 0)

    def pair_body(gg, carry):
        for h, (buf, sem) in enumerate(((out0, sem0), (out1, sem1))):
            b = base + 2 * gg + h

            @pl.when(gg >= 1)
            def _(buf=buf, sem=sem, b=b):
                pltpu.make_async_copy(buf, out_hbm.at[b], sem).wait()

            pltpu.sync_copy(x_hbm.at[b], x_v)
            compute(buf)
            pltpu.async_copy(buf, out_hbm.at[b], sem)
        return carry

    lax.fori_loop(0, B_PER_W // 2, pair_body, 0)
    pltpu.make_async_copy(out0, out_hbm.at[base], sem0).wait()
    pltpu.make_async_copy(out1, out_hbm.at[base], sem1).wait()


def kernel(x):
    return _pairwise(x)
